# Initial kernel scaffold; baseline (speedup 1.0000x reference)
import jax
import jax.numpy as jnp
from jax.experimental import pallas as pl


def kernel(x, edge_index, edge_weight, attention, W_z, b_z, LW_z, Lb_z, W_r, b_r, LW_r, Lb_r, W_h, b_h, LW_h, Lb_h, cls_W, cls_b):
    raise NotImplementedError("write your pallas kernel here")


# speedup vs baseline: 10.2313x; 10.2313x over previous
import functools

import jax
import jax.numpy as jnp
import numpy as np
from jax.experimental import pallas as pl
from jax.experimental.pallas import tpu as pltpu

_N = 50000
_E = 800000
_F = 8
_H = 32
_P = 12
_FT = _F * _P

_BLK = 1024
_NPAD = ((_N + _BLK - 1) // _BLK) * _BLK


def _dense_body(y_ref, x_ref, dis_ref, mzb_ref, mhb_ref, czb_ref, chb_ref,
                probs_ref, clsw_ref, clsb_ref, out_ref):
    dis = dis_ref[...]
    yt = y_ref[...] + (dis * dis) * x_ref[...]
    u_z = jnp.dot(yt, mzb_ref[...], preferred_element_type=jnp.float32) + czb_ref[...]
    u_h = jnp.dot(yt, mhb_ref[...], preferred_element_type=jnp.float32) + chb_ref[...]
    acc = jnp.zeros((_BLK, _H), dtype=jnp.float32)
    probs = probs_ref[...]
    for t in range(_P):
        z = jax.nn.sigmoid(u_z[:, t * _H:(t + 1) * _H])
        htil = jnp.tanh(u_h[:, t * _H:(t + 1) * _H])
        acc = acc + probs[0, t] * (1.0 - z) * htil
    h = jnp.maximum(acc, 0.0)
    out_ref[...] = jnp.dot(h, clsw_ref[...], preferred_element_type=jnp.float32) + clsb_ref[...]


@jax.jit
def _dense_stage(y_agg, x_flat, dis, mz_big, mh_big, cz_big, ch_big, probs, cls_W, cls_b):
    grid = (_NPAD // _BLK,)
    return pl.pallas_call(
        _dense_body,
        grid=grid,
        in_specs=[
            pl.BlockSpec((_BLK, _FT), lambda i: (i, 0)),
            pl.BlockSpec((_BLK, _FT), lambda i: (i, 0)),
            pl.BlockSpec((_BLK, 1), lambda i: (i, 0)),
            pl.BlockSpec((_FT, _P * _H), lambda i: (0, 0)),
            pl.BlockSpec((_FT, _P * _H), lambda i: (0, 0)),
            pl.BlockSpec((1, _P * _H), lambda i: (0, 0)),
            pl.BlockSpec((1, _P * _H), lambda i: (0, 0)),
            pl.BlockSpec((1, _P), lambda i: (0, 0)),
            pl.BlockSpec((_H, _P), lambda i: (0, 0)),
            pl.BlockSpec((1, _P), lambda i: (0, 0)),
        ],
        out_specs=pl.BlockSpec((_BLK, _P), lambda i: (i, 0)),
        out_shape=jax.ShapeDtypeStruct((_NPAD, _P), jnp.float32),
    )(y_agg, x_flat, dis, mz_big, mh_big, cz_big, ch_big, probs, cls_W, cls_b)


def kernel(x, edge_index, edge_weight, attention, W_z, b_z, LW_z, Lb_z,
           W_r, b_r, LW_r, Lb_r, W_h, b_h, LW_h, Lb_h, cls_W, cls_b):
    n = x.shape[0]
    src = edge_index[0]
    dst = edge_index[1]

    probs = jax.nn.softmax(attention)
    A_z = LW_z[:_H]
    A_h = LW_h[:_H]
    M_z = W_z @ A_z
    M_h = W_h @ A_h
    c_z = b_z @ A_z + Lb_z
    c_h = b_h @ A_h + Lb_h
    eye = jnp.eye(_P, dtype=jnp.float32)
    mz_big = jnp.einsum('fj,tu->ftuj', M_z, eye).reshape(_FT, _P * _H)
    mh_big = jnp.einsum('fj,tu->ftuj', M_h, eye).reshape(_FT, _P * _H)
    cz_big = jnp.tile(c_z, _P)[None, :]
    ch_big = jnp.tile(c_h, _P)[None, :]

    x_flat = x.reshape(n, _FT)

    deg = jax.ops.segment_sum(edge_weight, dst, num_segments=n) + 1.0
    deg_safe = jnp.where(deg > 0, deg, 1.0)
    dis = jnp.where(deg > 0, jax.lax.rsqrt(deg_safe), 0.0)

    norm = dis[src] * edge_weight * dis[dst]
    y_agg = jax.ops.segment_sum(norm[:, None] * x_flat[src], dst, num_segments=n)

    pad = _NPAD - n
    y_p = jnp.pad(y_agg, ((0, pad), (0, 0)))
    x_p = jnp.pad(x_flat, ((0, pad), (0, 0)))
    dis_p = jnp.pad(dis, (0, pad))[:, None]
    out = _dense_stage(y_p, x_p, dis_p, mz_big, mh_big, cz_big, ch_big,
                       probs[None, :], cls_W, cls_b[None, :])
    return out[:n]


# speedup vs baseline: 79.2761x; 7.7484x over previous
import functools

import jax
import jax.numpy as jnp
from jax import lax
from jax.experimental import pallas as pl
from jax.experimental.pallas import tpu as pltpu
from jax.experimental.pallas import tpu_sc as plsc

_N = 50000
_E = 800000
_F = 8
_H = 32
_P = 12
_FT = 96
_FH = 48
_NPC = 25088
_NPAD = 2 * _NPC
_YROWS = _NPC + 8
_DUMMY = _NPC
_TROWS = _NPC // 16
_DSL = _NPAD // 16
_ECH = 6272
_EROWS = _ECH + 8
_WCH_A = _ECH // 32
_TCH_C = _ECH // 16
_BLK = 1024

_mesh = plsc.VectorSubcoreMesh(core_axis_name="c", subcore_axis_name="s")


def _deg_body(dst_hbm, ew_hbm, out_hbm, deg_sh, dstv, ewv, zidx, zval, zero_v, ssem):
    c = lax.axis_index("c")
    s = lax.axis_index("s")
    w = c * 16 + s
    zeros16 = jnp.zeros((16,), jnp.float32)
    izeros16 = jnp.zeros((16,), jnp.int32)

    for g in range(8):
        zidx[0, pl.ds(g * 16, 16)] = izeros16
        zval[0, pl.ds(g * 16, 16)] = zeros16

    def _z(i, carry):
        zero_v[pl.ds(i * 16, 16)] = zeros16
        return carry
    lax.fori_loop(0, _DSL // 16, _z, 0)
    pltpu.sync_copy(zero_v, deg_sh.at[pl.ds(s * _DSL, _DSL)])
    plsc.subcore_barrier()

    base = w * _WCH_A

    def _step(i, carry):
        pltpu.sync_copy(dst_hbm.at[pl.ds(base + i * 4, 4)], dstv)
        pltpu.sync_copy(ew_hbm.at[pl.ds(base + i * 4, 4)], ewv)
        for q in range(4):
            pltpu.sync_copy(ewv.at[q], deg_sh.at[dstv.at[q]], add=True)
        return carry
    lax.fori_loop(0, _WCH_A // 4, _step, 0)

    plsc.subcore_barrier()
    pltpu.sync_copy(deg_sh.at[pl.ds(s * _DSL, _DSL)], zero_v)
    pltpu.sync_copy(zero_v, out_hbm.at[pl.ds(c * _NPAD + s * _DSL, _DSL)])


@functools.partial(
    pl.kernel,
    out_type=jax.ShapeDtypeStruct((2 * _NPAD,), jnp.float32),
    mesh=_mesh,
    scratch_types=[
        pltpu.VMEM_SHARED((_NPAD,), jnp.float32),
        pltpu.VMEM((4, 128), jnp.int32),
        pltpu.VMEM((4, 128), jnp.float32),
        pltpu.VMEM((1, 128), jnp.int32),
        pltpu.VMEM((1, 128), jnp.float32),
        pltpu.VMEM((_DSL,), jnp.float32),
        pltpu.SemaphoreType.DMA,
    ],
)
def _deg_kernel(dst_hbm, ew_hbm, out_hbm, deg_sh, dstv, ewv, zidx, zval, zero_v, ssem):
    _deg_body(dst_hbm, ew_hbm, out_hbm, deg_sh, dstv, ewv, zidx, zval, zero_v, ssem)


def _agg_body(src_hbm, dst_hbm, ew_hbm, xs0_hbm, xs1_hbm, out_hbm,
              y_sh, srcv, dstv, ewv, lidxv, rows, zidx, zrows,
              msem0, msem1, gsem0, gsem1, ssem0, ssem1):
    c = lax.axis_index("c")
    s = lax.axis_index("s")
    lo = c * _NPC
    msem = (msem0, msem1)
    gsem = (gsem0, gsem1)
    ssem = (ssem0, ssem1)
    zeros16 = jnp.zeros((16,), jnp.float32)
    izeros16 = jnp.zeros((16,), jnp.int32)
    iota16 = lax.iota(jnp.int32, 16)

    for g in range(8):
        zidx[0, pl.ds(g * 16, 16)] = izeros16
    for r in range(32):
        for f in range(3):
            zrows[r, pl.ds(f * 16, 16)] = zeros16

    tbase = s * _TCH_C

    for p in range(2):
        xs_hbm = xs0_hbm if p == 0 else xs1_hbm

        ybase = s * _TROWS
        for t in range(49):
            pltpu.sync_copy(zrows, y_sh.at[pl.ds(ybase + t * 32, 32)])

        @pl.when(s == 0)
        def _zero_dummy():
            pltpu.sync_copy(zrows.at[pl.ds(0, 8)],
                            y_sh.at[pl.ds(_DUMMY, 8)])

        plsc.subcore_barrier()

        def _iter(i, carry):
            for b in range(2):
                j = i * 2 + b
                pltpu.sync_copy(src_hbm.at[pl.ds(tbase + j * 2, 2)], srcv.at[b])
                pltpu.sync_copy(dst_hbm.at[pl.ds(tbase + j * 2, 2)], dstv.at[b])
                pltpu.sync_copy(ew_hbm.at[pl.ds(tbase + j * 2, 2)], ewv.at[b])
                gds = []
                for q in range(2):
                    gds.append(pltpu.async_copy(xs_hbm.at[srcv.at[b, q]],
                                                rows.at[b, q], gsem[b]))
                for q in range(2):
                    for g in range(8):
                        d16 = dstv[b, q, pl.ds(g * 16, 16)]
                        li = d16 - lo
                        ok = (li >= 0) & (li < _NPC)
                        lidxv[b, q, pl.ds(g * 16, 16)] = jnp.where(ok, li, _DUMMY)
                for d in gds:
                    d.wait()
                for q in range(2):
                    def _scale(g, carry2, _q=q, _b=b):
                        ew16 = ewv[_b, _q, pl.ds(g * 16, 16)]
                        for l in range(16):
                            k = g * 16 + l
                            w16 = lax.gather(
                                ew16, jnp.full((16, 1), l, jnp.int32),
                                lax.GatherDimensionNumbers(
                                    offset_dims=(), collapsed_slice_dims=(0,),
                                    start_index_map=(0,)),
                                (1,), mode=lax.GatherScatterMode.PROMISE_IN_BOUNDS)
                            for f in range(3):
                                sl = pl.ds(f * 16, 16)
                                rows[_b, _q, k, sl] = rows[_b, _q, k, sl] * w16
                        return carry2
                    lax.fori_loop(0, 8, _scale, 0)
                for q in range(2):
                    pltpu.sync_copy(rows.at[b, q], y_sh.at[lidxv.at[b, q]],
                                    add=True)
            return carry
        lax.fori_loop(0, _TCH_C // 4, _iter, 0)

        plsc.subcore_barrier()
        for t in range(12):
            pltpu.sync_copy(y_sh.at[pl.ds(ybase + t * 128, 128)], rows.at[0, 0])
            pltpu.sync_copy(rows.at[0, 0],
                            out_hbm.at[c, p, pl.ds(ybase + t * 128, 128)])
        pltpu.sync_copy(y_sh.at[pl.ds(ybase + 1536, 32)],
                        rows.at[0, 0, pl.ds(0, 32)])
        pltpu.sync_copy(rows.at[0, 0, pl.ds(0, 32)],
                        out_hbm.at[c, p, pl.ds(ybase + 1536, 32)])
        plsc.subcore_barrier()


@functools.partial(
    pl.kernel,
    out_type=jax.ShapeDtypeStruct((2, 2, _NPC, _FH), jnp.float32),
    mesh=_mesh,
    scratch_types=[
        pltpu.VMEM_SHARED((_YROWS, _FH), jnp.float32),
        pltpu.VMEM((2, 2, 128), jnp.int32),
        pltpu.VMEM((2, 2, 128), jnp.int32),
        pltpu.VMEM((2, 2, 128), jnp.float32),
        pltpu.VMEM((2, 2, 128), jnp.int32),
        pltpu.VMEM((2, 2, 128, _FH), jnp.float32),
        pltpu.VMEM((1, 128), jnp.int32),
        pltpu.VMEM((32, _FH), jnp.float32),
        pltpu.SemaphoreType.DMA,
        pltpu.SemaphoreType.DMA,
        pltpu.SemaphoreType.DMA,
        pltpu.SemaphoreType.DMA,
        pltpu.SemaphoreType.DMA,
        pltpu.SemaphoreType.DMA,
    ],
    compiler_params=pltpu.CompilerParams(use_tc_tiling_on_sc=False),
)
def _agg_kernel(src_hbm, dst_hbm, ew_hbm, xs0_hbm, xs1_hbm, out_hbm,
                y_sh, srcv, dstv, ewv, lidxv, rows, zidx, zrows,
                msem0, msem1, gsem0, gsem1, ssem0, ssem1):
    _agg_body(src_hbm, dst_hbm, ew_hbm, xs0_hbm, xs1_hbm, out_hbm,
              y_sh, srcv, dstv, ewv, lidxv, rows, zidx, zrows,
              msem0, msem1, gsem0, gsem1, ssem0, ssem1)


def _scale_body(degp_ref, xh0_ref, xh1_ref, xs0_ref, xs1_ref):
    d = degp_ref[0, :] + degp_ref[1, :] + 1.0
    d_safe = jnp.where(d > 0, d, 1.0)
    dis = jnp.where(d > 0, lax.rsqrt(d_safe), 0.0)[:, None]
    xs0_ref[...] = xh0_ref[...] * dis
    xs1_ref[...] = xh1_ref[...] * dis


def _prescale_stage(degp, xh0, xh1):
    return pl.pallas_call(
        _scale_body,
        grid=(_NPAD // _BLK,),
        in_specs=[
            pl.BlockSpec((2, _BLK), lambda i: (0, i)),
            pl.BlockSpec((_BLK, _FH), lambda i: (i, 0)),
            pl.BlockSpec((_BLK, _FH), lambda i: (i, 0)),
        ],
        out_specs=[
            pl.BlockSpec((_BLK, _FH), lambda i: (i, 0)),
            pl.BlockSpec((_BLK, _FH), lambda i: (i, 0)),
        ],
        out_shape=[
            jax.ShapeDtypeStruct((_NPAD, _FH), jnp.float32),
            jax.ShapeDtypeStruct((_NPAD, _FH), jnp.float32),
        ],
    )(degp, xh0, xh1)


def _dense_body(y_ref, xs0_ref, xs1_ref, degp_ref, mzb_ref, mhb_ref, czb_ref,
                chb_ref, probs_ref, clsw_ref, clsb_ref, out_ref):
    d = degp_ref[0, :] + degp_ref[1, :] + 1.0
    d_safe = jnp.where(d > 0, d, 1.0)
    dis = jnp.where(d > 0, lax.rsqrt(d_safe), 0.0)[:, None]
    xs = jnp.concatenate([xs0_ref[...], xs1_ref[...]], axis=1)
    yt = (y_ref[...] + xs) * dis
    u_z = jnp.dot(yt, mzb_ref[...], preferred_element_type=jnp.float32) + czb_ref[...]
    u_h = jnp.dot(yt, mhb_ref[...], preferred_element_type=jnp.float32) + chb_ref[...]
    acc = jnp.zeros((_BLK, _H), dtype=jnp.float32)
    probs = probs_ref[...]
    for t in range(_P):
        z = jax.nn.sigmoid(u_z[:, t * _H:(t + 1) * _H])
        htil = jnp.tanh(u_h[:, t * _H:(t + 1) * _H])
        acc = acc + probs[0, t] * (1.0 - z) * htil
    h = jnp.maximum(acc, 0.0)
    out_ref[...] = jnp.dot(h, clsw_ref[...], preferred_element_type=jnp.float32) + clsb_ref[...]


def _dense_stage(y_raw, xs0, xs1, degp, mz_big, mh_big, cz_big, ch_big, probs,
                 cls_W, cls_b):
    return pl.pallas_call(
        _dense_body,
        grid=(_NPAD // _BLK,),
        in_specs=[
            pl.BlockSpec((_BLK, _FT), lambda i: (i, 0)),
            pl.BlockSpec((_BLK, _FH), lambda i: (i, 0)),
            pl.BlockSpec((_BLK, _FH), lambda i: (i, 0)),
            pl.BlockSpec((2, _BLK), lambda i: (0, i)),
            pl.BlockSpec((_FT, _P * _H), lambda i: (0, 0)),
            pl.BlockSpec((_FT, _P * _H), lambda i: (0, 0)),
            pl.BlockSpec((1, _P * _H), lambda i: (0, 0)),
            pl.BlockSpec((1, _P * _H), lambda i: (0, 0)),
            pl.BlockSpec((1, _P), lambda i: (0, 0)),
            pl.BlockSpec((_H, _P), lambda i: (0, 0)),
            pl.BlockSpec((1, _P), lambda i: (0, 0)),
        ],
        out_specs=pl.BlockSpec((_BLK, _P), lambda i: (i, 0)),
        out_shape=jax.ShapeDtypeStruct((_NPAD, _P), jnp.float32),
    )(y_raw, xs0, xs1, degp, mz_big, mh_big, cz_big, ch_big, probs, cls_W, cls_b)


@jax.jit
def _run(x, edge_index, edge_weight, attention, W_z, b_z, LW_z, Lb_z,
         W_h, b_h, LW_h, Lb_h, cls_W, cls_b):
    n = x.shape[0]

    probs = jax.nn.softmax(attention)
    A_z = LW_z[:_H]
    A_h = LW_h[:_H]
    M_z = W_z @ A_z
    M_h = W_h @ A_h
    c_z = b_z @ A_z + Lb_z
    c_h = b_h @ A_h + Lb_h
    eye = jnp.eye(_P, dtype=jnp.float32)
    mz_big = jnp.einsum('fj,tu->ftuj', M_z, eye).reshape(_FT, _P * _H)
    mh_big = jnp.einsum('fj,tu->ftuj', M_h, eye).reshape(_FT, _P * _H)
    cz_big = jnp.tile(c_z, _P)[None, :]
    ch_big = jnp.tile(c_h, _P)[None, :]

    x_flat = x.reshape(n, _FT)
    x_p = jnp.pad(x_flat, ((0, _NPAD - n), (0, 0)))
    xh0 = x_p[:, :_FH]
    xh1 = x_p[:, _FH:]
    epad = _EROWS * 128 - _E
    src2d = jnp.pad(edge_index[0], (0, epad)).reshape(_EROWS, 128)
    dst2d = jnp.pad(edge_index[1], (0, epad)).reshape(_EROWS, 128)
    ew2d = jnp.pad(edge_weight, (0, epad)).reshape(_EROWS, 128)

    degp = _deg_kernel(dst2d, ew2d).reshape(2, _NPAD)

    xs0, xs1 = _prescale_stage(degp, xh0, xh1)

    yout = _agg_kernel(src2d, dst2d, ew2d, xs0, xs1)

    y_raw = yout.transpose(0, 2, 1, 3).reshape(_NPAD, _FT)

    out = _dense_stage(y_raw, xs0, xs1, degp, mz_big, mh_big, cz_big, ch_big,
                       probs[None, :], cls_W, cls_b[None, :])
    return out[:n]


def kernel(x, edge_index, edge_weight, attention, W_z, b_z, LW_z, Lb_z,
           W_r, b_r, LW_r, Lb_r, W_h, b_h, LW_h, Lb_h, cls_W, cls_b):
    return _run(x, edge_index, edge_weight, attention, W_z, b_z, LW_z, Lb_z,
                W_h, b_h, LW_h, Lb_h, cls_W, cls_b)


# speedup vs baseline: 100.2202x; 1.2642x over previous
import functools

import jax
import jax.numpy as jnp
from jax import lax
from jax.experimental import pallas as pl
from jax.experimental.pallas import tpu as pltpu
from jax.experimental.pallas import tpu_sc as plsc

_N = 50000
_E = 800000
_F = 8
_H = 32
_P = 12
_FT = 96
_FH = 48
_NPC = 25600
_NPAD = 2 * _NPC
_YROWS = _NPC + 8
_DUMMY = _NPC
_TROWS = _NPC // 16
_DSL = _NPAD // 16
_ECH = 6272
_EROWS = _ECH + 8
_WCH_A = _ECH // 32
_TCH_C = _ECH // 16
_BLK = 1024

_mesh = plsc.VectorSubcoreMesh(core_axis_name="c", subcore_axis_name="s")


def _deg_body(dst_hbm, ew_hbm, out_hbm, deg_sh, dstv, ewv, zidx, zval, zero_v, ssem):
    c = lax.axis_index("c")
    s = lax.axis_index("s")
    w = c * 16 + s
    zeros16 = jnp.zeros((16,), jnp.float32)
    izeros16 = jnp.zeros((16,), jnp.int32)

    for g in range(8):
        zidx[0, pl.ds(g * 16, 16)] = izeros16
        zval[0, pl.ds(g * 16, 16)] = zeros16

    def _z(i, carry):
        zero_v[pl.ds(i * 16, 16)] = zeros16
        return carry
    lax.fori_loop(0, _DSL // 16, _z, 0)
    pltpu.sync_copy(zero_v, deg_sh.at[pl.ds(s * _DSL, _DSL)])
    plsc.subcore_barrier()

    base = w * _WCH_A

    def _step(i, carry):
        pltpu.sync_copy(dst_hbm.at[pl.ds(base + i * 4, 4)], dstv)
        pltpu.sync_copy(ew_hbm.at[pl.ds(base + i * 4, 4)], ewv)
        for q in range(4):
            pltpu.sync_copy(ewv.at[q], deg_sh.at[dstv.at[q]], add=True)
        return carry
    lax.fori_loop(0, _WCH_A // 4, _step, 0)

    plsc.subcore_barrier()
    pltpu.sync_copy(deg_sh.at[pl.ds(s * _DSL, _DSL)], zero_v)
    pltpu.sync_copy(zero_v, out_hbm.at[pl.ds(c * _NPAD + s * _DSL, _DSL)])


@functools.partial(
    pl.kernel,
    out_type=jax.ShapeDtypeStruct((2 * _NPAD,), jnp.float32),
    mesh=_mesh,
    scratch_types=[
        pltpu.VMEM_SHARED((_NPAD,), jnp.float32),
        pltpu.VMEM((4, 128), jnp.int32),
        pltpu.VMEM((4, 128), jnp.float32),
        pltpu.VMEM((1, 128), jnp.int32),
        pltpu.VMEM((1, 128), jnp.float32),
        pltpu.VMEM((_DSL,), jnp.float32),
        pltpu.SemaphoreType.DMA,
    ],
)
def _deg_kernel(dst_hbm, ew_hbm, out_hbm, deg_sh, dstv, ewv, zidx, zval, zero_v, ssem):
    _deg_body(dst_hbm, ew_hbm, out_hbm, deg_sh, dstv, ewv, zidx, zval, zero_v, ssem)


def _agg_body(meta_hbm, ew_hbm, xs0_hbm, xs1_hbm, out_hbm,
              y_sh, mv, ewb, lidxv, rows, zrows, gsemA, gsemB, ssem):
    c = lax.axis_index("c")
    s = lax.axis_index("s")
    lo = c * _NPC
    zeros16 = jnp.zeros((16,), jnp.float32)

    for r in range(32):
        for f in range(3):
            zrows[r, pl.ds(f * 16, 16)] = zeros16

    tbase = s * _TCH_C

    for p in range(2):
        xs_hbm = xs0_hbm if p == 0 else xs1_hbm

        ybase = s * _TROWS
        for t in range(_TROWS // 32):
            pltpu.sync_copy(zrows, y_sh.at[pl.ds(ybase + t * 32, 32)])

        @pl.when(s == 0)
        def _zero_dummy():
            pltpu.sync_copy(zrows.at[pl.ds(0, 8)],
                            y_sh.at[pl.ds(_DUMMY, 8)])

        plsc.subcore_barrier()

        def _iter(i, carry):
            pltpu.sync_copy(meta_hbm.at[pl.ds(tbase + i * 4, 4)], mv)
            pltpu.sync_copy(ew_hbm.at[pl.ds(tbase + i * 4, 4)], ewb)
            gds = []
            for q in range(4):
                gds.append(pltpu.async_copy(xs_hbm.at[mv.at[q, 0]],
                                            rows.at[q],
                                            gsemA if q < 2 else gsemB))
            for q in range(4):
                for g in range(8):
                    d16 = mv[q, 1, pl.ds(g * 16, 16)]
                    li = d16 - lo
                    ok = (li >= 0) & (li < _NPC)
                    lidxv[q, pl.ds(g * 16, 16)] = jnp.where(ok, li, _DUMMY)
            sds = []
            for half in range(2):
                for q in (2 * half, 2 * half + 1):
                    gds[q].wait()
                for q in (2 * half, 2 * half + 1):
                    def _scale(g, carry2, _q=q):
                        ew16 = ewb[_q, pl.ds(g * 16, 16)]
                        for l in range(16):
                            k = g * 16 + l
                            w16 = lax.gather(
                                ew16, jnp.full((16, 1), l, jnp.int32),
                                lax.GatherDimensionNumbers(
                                    offset_dims=(), collapsed_slice_dims=(0,),
                                    start_index_map=(0,)),
                                (1,),
                                mode=lax.GatherScatterMode.PROMISE_IN_BOUNDS)
                            for f in range(3):
                                sl = pl.ds(f * 16, 16)
                                rows[_q, k, sl] = rows[_q, k, sl] * w16
                        return carry2
                    lax.fori_loop(0, 8, _scale, 0)
                for q in (2 * half, 2 * half + 1):
                    sds.append(pltpu.async_copy(rows.at[q],
                                                y_sh.at[lidxv.at[q]],
                                                ssem, add=True))
            for d in sds:
                d.wait()
            return carry
        lax.fori_loop(0, _TCH_C // 4, _iter, 0)

        plsc.subcore_barrier()
        for t in range(12):
            pltpu.sync_copy(y_sh.at[pl.ds(ybase + t * 128, 128)], rows.at[0])
            pltpu.sync_copy(rows.at[0],
                            out_hbm.at[c, p, pl.ds(ybase + t * 128, 128)])
        pltpu.sync_copy(y_sh.at[pl.ds(ybase + 1536, 64)],
                        rows.at[0, pl.ds(0, 64)])
        pltpu.sync_copy(rows.at[0, pl.ds(0, 64)],
                        out_hbm.at[c, p, pl.ds(ybase + 1536, 64)])
        plsc.subcore_barrier()


@functools.partial(
    pl.kernel,
    out_type=jax.ShapeDtypeStruct((2, 2, _NPC, _FH), jnp.float32),
    mesh=_mesh,
    scratch_types=[
        pltpu.VMEM_SHARED((_YROWS, _FH), jnp.float32),
        pltpu.VMEM((4, 2, 128), jnp.int32),
        pltpu.VMEM((4, 128), jnp.float32),
        pltpu.VMEM((4, 128), jnp.int32),
        pltpu.VMEM((4, 128, _FH), jnp.float32),
        pltpu.VMEM((32, _FH), jnp.float32),
        pltpu.SemaphoreType.DMA,
        pltpu.SemaphoreType.DMA,
        pltpu.SemaphoreType.DMA,
    ],
    compiler_params=pltpu.CompilerParams(use_tc_tiling_on_sc=False),
)
def _agg_kernel(meta_hbm, ew_hbm, xs0_hbm, xs1_hbm, out_hbm,
                y_sh, mv, ewb, lidxv, rows, zrows, gsemA, gsemB, ssem):
    _agg_body(meta_hbm, ew_hbm, xs0_hbm, xs1_hbm, out_hbm,
              y_sh, mv, ewb, lidxv, rows, zrows, gsemA, gsemB, ssem)


def _scale_body(degp_ref, xh0_ref, xh1_ref, xs0_ref, xs1_ref):
    d = degp_ref[0, :] + degp_ref[1, :] + 1.0
    d_safe = jnp.where(d > 0, d, 1.0)
    dis = jnp.where(d > 0, lax.rsqrt(d_safe), 0.0)[:, None]
    xs0_ref[...] = xh0_ref[...] * dis
    xs1_ref[...] = xh1_ref[...] * dis


def _prescale_stage(degp, xh0, xh1):
    return pl.pallas_call(
        _scale_body,
        grid=(_NPAD // _BLK,),
        in_specs=[
            pl.BlockSpec((2, _BLK), lambda i: (0, i)),
            pl.BlockSpec((_BLK, _FH), lambda i: (i, 0)),
            pl.BlockSpec((_BLK, _FH), lambda i: (i, 0)),
        ],
        out_specs=[
            pl.BlockSpec((_BLK, _FH), lambda i: (i, 0)),
            pl.BlockSpec((_BLK, _FH), lambda i: (i, 0)),
        ],
        out_shape=[
            jax.ShapeDtypeStruct((_NPAD, _FH), jnp.float32),
            jax.ShapeDtypeStruct((_NPAD, _FH), jnp.float32),
        ],
    )(degp, xh0, xh1)


def _dense_body(y_ref, xs0_ref, xs1_ref, degp_ref, mzb_ref, mhb_ref, czb_ref,
                chb_ref, probs_ref, clsw_ref, clsb_ref, out_ref):
    d = degp_ref[0, :] + degp_ref[1, :] + 1.0
    d_safe = jnp.where(d > 0, d, 1.0)
    dis = jnp.where(d > 0, lax.rsqrt(d_safe), 0.0)[:, None]
    xs = jnp.concatenate([xs0_ref[...], xs1_ref[...]], axis=1)
    y = jnp.concatenate([y_ref[0, 0], y_ref[0, 1]], axis=1)
    yt = (y + xs) * dis
    u_z = jnp.dot(yt, mzb_ref[...], preferred_element_type=jnp.float32) + czb_ref[...]
    u_h = jnp.dot(yt, mhb_ref[...], preferred_element_type=jnp.float32) + chb_ref[...]
    acc = jnp.zeros((_BLK, _H), dtype=jnp.float32)
    probs = probs_ref[...]
    for t in range(_P):
        z = jax.nn.sigmoid(u_z[:, t * _H:(t + 1) * _H])
        htil = jnp.tanh(u_h[:, t * _H:(t + 1) * _H])
        acc = acc + probs[0, t] * (1.0 - z) * htil
    h = jnp.maximum(acc, 0.0)
    out_ref[...] = jnp.dot(h, clsw_ref[...], preferred_element_type=jnp.float32) + clsb_ref[...]


def _dense_stage(y_raw, xs0, xs1, degp, mz_big, mh_big, cz_big, ch_big, probs,
                 cls_W, cls_b):
    return pl.pallas_call(
        _dense_body,
        grid=(_NPAD // _BLK,),
        in_specs=[
            pl.BlockSpec((1, 2, _BLK, _FH), lambda i: (i // 25, 0, i % 25, 0)),
            pl.BlockSpec((_BLK, _FH), lambda i: (i, 0)),
            pl.BlockSpec((_BLK, _FH), lambda i: (i, 0)),
            pl.BlockSpec((2, _BLK), lambda i: (0, i)),
            pl.BlockSpec((_FT, _P * _H), lambda i: (0, 0)),
            pl.BlockSpec((_FT, _P * _H), lambda i: (0, 0)),
            pl.BlockSpec((1, _P * _H), lambda i: (0, 0)),
            pl.BlockSpec((1, _P * _H), lambda i: (0, 0)),
            pl.BlockSpec((1, _P), lambda i: (0, 0)),
            pl.BlockSpec((_H, _P), lambda i: (0, 0)),
            pl.BlockSpec((1, _P), lambda i: (0, 0)),
        ],
        out_specs=pl.BlockSpec((_BLK, _P), lambda i: (i, 0)),
        out_shape=jax.ShapeDtypeStruct((_NPAD, _P), jnp.float32),
    )(y_raw, xs0, xs1, degp, mz_big, mh_big, cz_big, ch_big, probs, cls_W, cls_b)


@jax.jit
def _run(x, edge_index, edge_weight, attention, W_z, b_z, LW_z, Lb_z,
         W_h, b_h, LW_h, Lb_h, cls_W, cls_b):
    n = x.shape[0]

    probs = jax.nn.softmax(attention)
    A_z = LW_z[:_H]
    A_h = LW_h[:_H]
    M_z = W_z @ A_z
    M_h = W_h @ A_h
    c_z = b_z @ A_z + Lb_z
    c_h = b_h @ A_h + Lb_h
    eye = jnp.eye(_P, dtype=jnp.float32)
    mz_big = jnp.einsum('fj,tu->ftuj', M_z, eye).reshape(_FT, _P * _H)
    mh_big = jnp.einsum('fj,tu->ftuj', M_h, eye).reshape(_FT, _P * _H)
    cz_big = jnp.tile(c_z, _P)[None, :]
    ch_big = jnp.tile(c_h, _P)[None, :]

    x_flat = x.reshape(n, _FT)
    x_p = jnp.pad(x_flat, ((0, _NPAD - n), (0, 0)))
    xh0 = x_p[:, :_FH]
    xh1 = x_p[:, _FH:]
    epad = _EROWS * 128 - _E
    src2d = jnp.pad(edge_index[0], (0, epad)).reshape(_EROWS, 128)
    dst2d = jnp.pad(edge_index[1], (0, epad)).reshape(_EROWS, 128)
    ew2d = jnp.pad(edge_weight, (0, epad)).reshape(_EROWS, 128)
    meta = jnp.stack([src2d, dst2d], axis=1)

    degp = _deg_kernel(dst2d, ew2d).reshape(2, _NPAD)

    xs0, xs1 = _prescale_stage(degp, xh0, xh1)

    yout = _agg_kernel(meta, ew2d, xs0, xs1)

    out = _dense_stage(yout, xs0, xs1, degp, mz_big, mh_big, cz_big, ch_big,
                       probs[None, :], cls_W, cls_b[None, :])
    return out[:n]


def kernel(x, edge_index, edge_weight, attention, W_z, b_z, LW_z, Lb_z,
           W_r, b_r, LW_r, Lb_r, W_h, b_h, LW_h, Lb_h, cls_W, cls_b):
    return _run(x, edge_index, edge_weight, attention, W_z, b_z, LW_z, Lb_z,
                W_h, b_h, LW_h, Lb_h, cls_W, cls_b)


# speedup vs baseline: 101.1122x; 1.0089x over previous
import functools

import jax
import jax.numpy as jnp
from jax import lax
from jax.experimental import pallas as pl
from jax.experimental.pallas import tpu as pltpu
from jax.experimental.pallas import tpu_sc as plsc

_N = 50000
_E = 800000
_F = 8
_H = 32
_P = 12
_FT = 96
_FH = 48
_NPC = 25600
_NPAD = 2 * _NPC
_YROWS = _NPC + 8
_DUMMY = _NPC
_TROWS = _NPC // 16
_DSL = _NPAD // 16
_ECH = 6272
_EROWS = _ECH + 8
_WCH_A = _ECH // 32
_TCH_C = _ECH // 16
_BLK = 1024

_mesh = plsc.VectorSubcoreMesh(core_axis_name="c", subcore_axis_name="s")


def _deg_body(dst_hbm, ew_hbm, out_hbm, deg_sh, dstv, ewv, zidx, zval, zero_v, ssem):
    c = lax.axis_index("c")
    s = lax.axis_index("s")
    w = c * 16 + s
    zeros16 = jnp.zeros((16,), jnp.float32)
    izeros16 = jnp.zeros((16,), jnp.int32)

    for g in range(8):
        zidx[0, pl.ds(g * 16, 16)] = izeros16
        zval[0, pl.ds(g * 16, 16)] = zeros16

    def _z(i, carry):
        zero_v[pl.ds(i * 16, 16)] = zeros16
        return carry
    lax.fori_loop(0, _DSL // 16, _z, 0)
    pltpu.sync_copy(zero_v, deg_sh.at[pl.ds(s * _DSL, _DSL)])
    plsc.subcore_barrier()

    base = w * _WCH_A

    def _step(i, carry):
        pltpu.sync_copy(dst_hbm.at[pl.ds(base + i * 4, 4)], dstv)
        pltpu.sync_copy(ew_hbm.at[pl.ds(base + i * 4, 4)], ewv)
        for q in range(4):
            pltpu.sync_copy(ewv.at[q], deg_sh.at[dstv.at[q]], add=True)
        return carry
    lax.fori_loop(0, _WCH_A // 4, _step, 0)

    plsc.subcore_barrier()
    pltpu.sync_copy(deg_sh.at[pl.ds(s * _DSL, _DSL)], zero_v)
    pltpu.sync_copy(zero_v, out_hbm.at[pl.ds(c * _NPAD + s * _DSL, _DSL)])


@functools.partial(
    pl.kernel,
    out_type=jax.ShapeDtypeStruct((2 * _NPAD,), jnp.float32),
    mesh=_mesh,
    scratch_types=[
        pltpu.VMEM_SHARED((_NPAD,), jnp.float32),
        pltpu.VMEM((4, 128), jnp.int32),
        pltpu.VMEM((4, 128), jnp.float32),
        pltpu.VMEM((1, 128), jnp.int32),
        pltpu.VMEM((1, 128), jnp.float32),
        pltpu.VMEM((_DSL,), jnp.float32),
        pltpu.SemaphoreType.DMA,
    ],
)
def _deg_kernel(dst_hbm, ew_hbm, out_hbm, deg_sh, dstv, ewv, zidx, zval, zero_v, ssem):
    _deg_body(dst_hbm, ew_hbm, out_hbm, deg_sh, dstv, ewv, zidx, zval, zero_v, ssem)


def _agg_body(meta_hbm, ew_hbm, xs0_hbm, xs1_hbm, out_hbm,
              y_sh, mv, ewb, lidxv, rows, zrows, gsemA, gsemB, ssem,
              msem0, msem1):
    c = lax.axis_index("c")
    s = lax.axis_index("s")
    lo = c * _NPC
    zeros16 = jnp.zeros((16,), jnp.float32)

    for r in range(32):
        for f in range(3):
            zrows[r, pl.ds(f * 16, 16)] = zeros16

    tbase = s * _TCH_C

    for p in range(2):
        xs_hbm = xs0_hbm if p == 0 else xs1_hbm

        ybase = s * _TROWS
        for t in range(_TROWS // 32):
            pltpu.sync_copy(zrows, y_sh.at[pl.ds(ybase + t * 32, 32)])

        @pl.when(s == 0)
        def _zero_dummy():
            pltpu.sync_copy(zrows.at[pl.ds(0, 8)],
                            y_sh.at[pl.ds(_DUMMY, 8)])

        plsc.subcore_barrier()

        pltpu.async_copy(meta_hbm.at[pl.ds(tbase, 4)], mv.at[0], msem0)
        pltpu.async_copy(ew_hbm.at[pl.ds(tbase, 4)], ewb.at[0], msem0)
        pltpu.async_copy(meta_hbm.at[pl.ds(tbase + 4, 4)], mv.at[1], msem1)
        pltpu.async_copy(ew_hbm.at[pl.ds(tbase + 4, 4)], ewb.at[1], msem1)

        def _iter(i, carry):
            for b in range(2):
                sem = msem0 if b == 0 else msem1
                j = i * 2 + b
                pltpu.make_async_copy(meta_hbm.at[pl.ds(0, 4)], mv.at[b],
                                      sem).wait()
                pltpu.make_async_copy(ew_hbm.at[pl.ds(0, 4)], ewb.at[b],
                                      sem).wait()
                gds = []
                for q in range(4):
                    gds.append(pltpu.async_copy(xs_hbm.at[mv.at[b, q, 0]],
                                                rows.at[q],
                                                gsemA if q < 2 else gsemB))
                for q in range(4):
                    for g in range(8):
                        d16 = mv[b, q, 1, pl.ds(g * 16, 16)]
                        li = d16 - lo
                        ok = (li >= 0) & (li < _NPC)
                        lidxv[q, pl.ds(g * 16, 16)] = jnp.where(ok, li, _DUMMY)
                sds = []
                for half in range(2):
                    for q in (2 * half, 2 * half + 1):
                        gds[q].wait()
                    for q in (2 * half, 2 * half + 1):
                        def _scale(g, carry2, _q=q, _b=b):
                            ew16 = ewb[_b, _q, pl.ds(g * 16, 16)]
                            for l in range(16):
                                k = g * 16 + l
                                w16 = lax.gather(
                                    ew16, jnp.full((16, 1), l, jnp.int32),
                                    lax.GatherDimensionNumbers(
                                        offset_dims=(),
                                        collapsed_slice_dims=(0,),
                                        start_index_map=(0,)),
                                    (1,),
                                    mode=lax.GatherScatterMode.PROMISE_IN_BOUNDS)
                                for f in range(3):
                                    sl = pl.ds(f * 16, 16)
                                    rows[_q, k, sl] = rows[_q, k, sl] * w16
                            return carry2
                        lax.fori_loop(0, 8, _scale, 0)
                    for q in (2 * half, 2 * half + 1):
                        sds.append(pltpu.async_copy(rows.at[q],
                                                    y_sh.at[lidxv.at[q]],
                                                    ssem, add=True))
                nb = tbase + (j + 2) * 4
                pltpu.async_copy(meta_hbm.at[pl.ds(nb, 4)], mv.at[b], sem)
                pltpu.async_copy(ew_hbm.at[pl.ds(nb, 4)], ewb.at[b], sem)
                for d in sds:
                    d.wait()
            return carry
        lax.fori_loop(0, _TCH_C // 8, _iter, 0)

        pltpu.make_async_copy(meta_hbm.at[pl.ds(0, 4)], mv.at[0], msem0).wait()
        pltpu.make_async_copy(ew_hbm.at[pl.ds(0, 4)], ewb.at[0], msem0).wait()
        pltpu.make_async_copy(meta_hbm.at[pl.ds(0, 4)], mv.at[1], msem1).wait()
        pltpu.make_async_copy(ew_hbm.at[pl.ds(0, 4)], ewb.at[1], msem1).wait()

        plsc.subcore_barrier()
        for t in range(12):
            pltpu.sync_copy(y_sh.at[pl.ds(ybase + t * 128, 128)], rows.at[0])
            pltpu.sync_copy(rows.at[0],
                            out_hbm.at[c, p, pl.ds(ybase + t * 128, 128)])
        pltpu.sync_copy(y_sh.at[pl.ds(ybase + 1536, 64)],
                        rows.at[0, pl.ds(0, 64)])
        pltpu.sync_copy(rows.at[0, pl.ds(0, 64)],
                        out_hbm.at[c, p, pl.ds(ybase + 1536, 64)])
        plsc.subcore_barrier()


@functools.partial(
    pl.kernel,
    out_type=jax.ShapeDtypeStruct((2, 2, _NPC, _FH), jnp.float32),
    mesh=_mesh,
    scratch_types=[
        pltpu.VMEM_SHARED((_YROWS, _FH), jnp.float32),
        pltpu.VMEM((2, 4, 2, 128), jnp.int32),
        pltpu.VMEM((2, 4, 128), jnp.float32),
        pltpu.VMEM((4, 128), jnp.int32),
        pltpu.VMEM((4, 128, _FH), jnp.float32),
        pltpu.VMEM((32, _FH), jnp.float32),
        pltpu.SemaphoreType.DMA,
        pltpu.SemaphoreType.DMA,
        pltpu.SemaphoreType.DMA,
        pltpu.SemaphoreType.DMA,
        pltpu.SemaphoreType.DMA,
    ],
    compiler_params=pltpu.CompilerParams(use_tc_tiling_on_sc=False),
)
def _agg_kernel(meta_hbm, ew_hbm, xs0_hbm, xs1_hbm, out_hbm,
                y_sh, mv, ewb, lidxv, rows, zrows, gsemA, gsemB, ssem,
                msem0, msem1):
    _agg_body(meta_hbm, ew_hbm, xs0_hbm, xs1_hbm, out_hbm,
              y_sh, mv, ewb, lidxv, rows, zrows, gsemA, gsemB, ssem,
              msem0, msem1)


def _scale_body(degp_ref, xh0_ref, xh1_ref, xs0_ref, xs1_ref):
    d = degp_ref[0, :] + degp_ref[1, :] + 1.0
    d_safe = jnp.where(d > 0, d, 1.0)
    dis = jnp.where(d > 0, lax.rsqrt(d_safe), 0.0)[:, None]
    xs0_ref[...] = xh0_ref[...] * dis
    xs1_ref[...] = xh1_ref[...] * dis


def _prescale_stage(degp, xh0, xh1):
    return pl.pallas_call(
        _scale_body,
        grid=(_NPAD // _BLK,),
        in_specs=[
            pl.BlockSpec((2, _BLK), lambda i: (0, i)),
            pl.BlockSpec((_BLK, _FH), lambda i: (i, 0)),
            pl.BlockSpec((_BLK, _FH), lambda i: (i, 0)),
        ],
        out_specs=[
            pl.BlockSpec((_BLK, _FH), lambda i: (i, 0)),
            pl.BlockSpec((_BLK, _FH), lambda i: (i, 0)),
        ],
        out_shape=[
            jax.ShapeDtypeStruct((_NPAD, _FH), jnp.float32),
            jax.ShapeDtypeStruct((_NPAD, _FH), jnp.float32),
        ],
    )(degp, xh0, xh1)


def _dense_body(y_ref, xs0_ref, xs1_ref, degp_ref, mzb_ref, mhb_ref, czb_ref,
                chb_ref, probs_ref, clsw_ref, clsb_ref, out_ref):
    d = degp_ref[0, :] + degp_ref[1, :] + 1.0
    d_safe = jnp.where(d > 0, d, 1.0)
    dis = jnp.where(d > 0, lax.rsqrt(d_safe), 0.0)[:, None]
    xs = jnp.concatenate([xs0_ref[...], xs1_ref[...]], axis=1)
    y = jnp.concatenate([y_ref[0, 0], y_ref[0, 1]], axis=1)
    yt = (y + xs) * dis
    u_z = jnp.dot(yt, mzb_ref[...], preferred_element_type=jnp.float32) + czb_ref[...]
    u_h = jnp.dot(yt, mhb_ref[...], preferred_element_type=jnp.float32) + chb_ref[...]
    acc = jnp.zeros((_BLK, _H), dtype=jnp.float32)
    probs = probs_ref[...]
    for t in range(_P):
        z = jax.nn.sigmoid(u_z[:, t * _H:(t + 1) * _H])
        htil = jnp.tanh(u_h[:, t * _H:(t + 1) * _H])
        acc = acc + probs[0, t] * (1.0 - z) * htil
    h = jnp.maximum(acc, 0.0)
    out_ref[...] = jnp.dot(h, clsw_ref[...], preferred_element_type=jnp.float32) + clsb_ref[...]


def _dense_stage(y_raw, xs0, xs1, degp, mz_big, mh_big, cz_big, ch_big, probs,
                 cls_W, cls_b):
    return pl.pallas_call(
        _dense_body,
        grid=(_NPAD // _BLK,),
        in_specs=[
            pl.BlockSpec((1, 2, _BLK, _FH), lambda i: (i // 25, 0, i % 25, 0)),
            pl.BlockSpec((_BLK, _FH), lambda i: (i, 0)),
            pl.BlockSpec((_BLK, _FH), lambda i: (i, 0)),
            pl.BlockSpec((2, _BLK), lambda i: (0, i)),
            pl.BlockSpec((_FT, _P * _H), lambda i: (0, 0)),
            pl.BlockSpec((_FT, _P * _H), lambda i: (0, 0)),
            pl.BlockSpec((1, _P * _H), lambda i: (0, 0)),
            pl.BlockSpec((1, _P * _H), lambda i: (0, 0)),
            pl.BlockSpec((1, _P), lambda i: (0, 0)),
            pl.BlockSpec((_H, _P), lambda i: (0, 0)),
            pl.BlockSpec((1, _P), lambda i: (0, 0)),
        ],
        out_specs=pl.BlockSpec((_BLK, _P), lambda i: (i, 0)),
        out_shape=jax.ShapeDtypeStruct((_NPAD, _P), jnp.float32),
    )(y_raw, xs0, xs1, degp, mz_big, mh_big, cz_big, ch_big, probs, cls_W, cls_b)


@jax.jit
def _run(x, edge_index, edge_weight, attention, W_z, b_z, LW_z, Lb_z,
         W_h, b_h, LW_h, Lb_h, cls_W, cls_b):
    n = x.shape[0]

    probs = jax.nn.softmax(attention)
    A_z = LW_z[:_H]
    A_h = LW_h[:_H]
    M_z = W_z @ A_z
    M_h = W_h @ A_h
    c_z = b_z @ A_z + Lb_z
    c_h = b_h @ A_h + Lb_h
    eye = jnp.eye(_P, dtype=jnp.float32)
    mz_big = jnp.einsum('fj,tu->ftuj', M_z, eye).reshape(_FT, _P * _H)
    mh_big = jnp.einsum('fj,tu->ftuj', M_h, eye).reshape(_FT, _P * _H)
    cz_big = jnp.tile(c_z, _P)[None, :]
    ch_big = jnp.tile(c_h, _P)[None, :]

    x_flat = x.reshape(n, _FT)
    x_p = jnp.pad(x_flat, ((0, _NPAD - n), (0, 0)))
    xh0 = x_p[:, :_FH]
    xh1 = x_p[:, _FH:]
    epad = _EROWS * 128 - _E
    src2d = jnp.pad(edge_index[0], (0, epad)).reshape(_EROWS, 128)
    dst2d = jnp.pad(edge_index[1], (0, epad)).reshape(_EROWS, 128)
    ew2d = jnp.pad(edge_weight, (0, epad)).reshape(_EROWS, 128)
    meta = jnp.stack([src2d, dst2d], axis=1)

    degp = _deg_kernel(dst2d, ew2d).reshape(2, _NPAD)

    xs0, xs1 = _prescale_stage(degp, xh0, xh1)

    yout = _agg_kernel(meta, ew2d, xs0, xs1)

    out = _dense_stage(yout, xs0, xs1, degp, mz_big, mh_big, cz_big, ch_big,
                       probs[None, :], cls_W, cls_b[None, :])
    return out[:n]


def kernel(x, edge_index, edge_weight, attention, W_z, b_z, LW_z, Lb_z,
           W_r, b_r, LW_r, Lb_r, W_h, b_h, LW_h, Lb_h, cls_W, cls_b):
    return _run(x, edge_index, edge_weight, attention, W_z, b_z, LW_z, Lb_z,
                W_h, b_h, LW_h, Lb_h, cls_W, cls_b)


# speedup vs baseline: 101.3361x; 1.0022x over previous
import functools

import jax
import jax.numpy as jnp
from jax import lax
from jax.experimental import pallas as pl
from jax.experimental.pallas import tpu as pltpu
from jax.experimental.pallas import tpu_sc as plsc

_N = 50000
_E = 800000
_F = 8
_H = 32
_P = 12
_FT = 96
_FH = 48
_NPC = 25600
_NPAD = 2 * _NPC
_YROWS = _NPC + 8
_DUMMY = _NPC
_TROWS = _NPC // 16
_DSL = _NPAD // 16
_ECH = 6272
_EROWS = _ECH + 8
_WCH_A = _ECH // 32
_TCH_C = _ECH // 16
_BLK = 1024

_mesh = plsc.VectorSubcoreMesh(core_axis_name="c", subcore_axis_name="s")


def _deg_body(dst_hbm, ew_hbm, out_hbm, deg_sh, dstv, ewv, zidx, zval, zero_v, ssem):
    c = lax.axis_index("c")
    s = lax.axis_index("s")
    w = c * 16 + s
    zeros16 = jnp.zeros((16,), jnp.float32)
    izeros16 = jnp.zeros((16,), jnp.int32)

    for g in range(8):
        zidx[0, pl.ds(g * 16, 16)] = izeros16
        zval[0, pl.ds(g * 16, 16)] = zeros16

    def _z(i, carry):
        zero_v[pl.ds(i * 16, 16)] = zeros16
        return carry
    lax.fori_loop(0, _DSL // 16, _z, 0)
    pltpu.sync_copy(zero_v, deg_sh.at[pl.ds(s * _DSL, _DSL)])
    plsc.subcore_barrier()

    base = w * _WCH_A

    def _step(i, carry):
        pltpu.sync_copy(dst_hbm.at[pl.ds(base + i * 4, 4)], dstv)
        pltpu.sync_copy(ew_hbm.at[pl.ds(base + i * 4, 4)], ewv)
        for q in range(4):
            pltpu.sync_copy(ewv.at[q], deg_sh.at[dstv.at[q]], add=True)
        return carry
    lax.fori_loop(0, _WCH_A // 4, _step, 0)

    plsc.subcore_barrier()
    pltpu.sync_copy(deg_sh.at[pl.ds(s * _DSL, _DSL)], zero_v)
    pltpu.sync_copy(zero_v, out_hbm.at[pl.ds(c * _NPAD + s * _DSL, _DSL)])


@functools.partial(
    pl.kernel,
    out_type=jax.ShapeDtypeStruct((2 * _NPAD,), jnp.float32),
    mesh=_mesh,
    scratch_types=[
        pltpu.VMEM_SHARED((_NPAD,), jnp.float32),
        pltpu.VMEM((4, 128), jnp.int32),
        pltpu.VMEM((4, 128), jnp.float32),
        pltpu.VMEM((1, 128), jnp.int32),
        pltpu.VMEM((1, 128), jnp.float32),
        pltpu.VMEM((_DSL,), jnp.float32),
        pltpu.SemaphoreType.DMA,
    ],
)
def _deg_kernel(dst_hbm, ew_hbm, out_hbm, deg_sh, dstv, ewv, zidx, zval, zero_v, ssem):
    _deg_body(dst_hbm, ew_hbm, out_hbm, deg_sh, dstv, ewv, zidx, zval, zero_v, ssem)


def _agg_body(src_hbm, dst_hbm, ew_hbm, xs0_hbm, xs1_hbm, out_hbm,
              y_sh, sv, dv, ewb, lidxv, rows, zrows, gsemA, gsemB, ssem,
              msem0, msem1):
    c = lax.axis_index("c")
    s = lax.axis_index("s")
    lo = c * _NPC
    zeros16 = jnp.zeros((16,), jnp.float32)

    for r in range(32):
        for f in range(3):
            zrows[r, pl.ds(f * 16, 16)] = zeros16

    tbase = s * _TCH_C

    for p in range(2):
        xs_hbm = xs0_hbm if p == 0 else xs1_hbm

        ybase = s * _TROWS
        for t in range(_TROWS // 32):
            pltpu.sync_copy(zrows, y_sh.at[pl.ds(ybase + t * 32, 32)])

        @pl.when(s == 0)
        def _zero_dummy():
            pltpu.sync_copy(zrows.at[pl.ds(0, 8)],
                            y_sh.at[pl.ds(_DUMMY, 8)])

        plsc.subcore_barrier()

        for b in range(2):
            semb = msem0 if b == 0 else msem1
            pltpu.async_copy(src_hbm.at[pl.ds(tbase + b * 4, 4)], sv.at[b], semb)
            pltpu.async_copy(dst_hbm.at[pl.ds(tbase + b * 4, 4)], dv.at[b], semb)
            pltpu.async_copy(ew_hbm.at[pl.ds(tbase + b * 4, 4)], ewb.at[b], semb)

        def _iter(i, carry):
            for b in range(2):
                sem = msem0 if b == 0 else msem1
                j = i * 2 + b
                pltpu.make_async_copy(src_hbm.at[pl.ds(0, 4)], sv.at[b],
                                      sem).wait()
                pltpu.make_async_copy(dst_hbm.at[pl.ds(0, 4)], dv.at[b],
                                      sem).wait()
                pltpu.make_async_copy(ew_hbm.at[pl.ds(0, 4)], ewb.at[b],
                                      sem).wait()
                gds = []
                for q in range(4):
                    gds.append(pltpu.async_copy(xs_hbm.at[sv.at[b, q]],
                                                rows.at[q],
                                                gsemA if q < 2 else gsemB))
                for q in range(4):
                    for g in range(8):
                        d16 = dv[b, q, pl.ds(g * 16, 16)]
                        li = d16 - lo
                        ok = (li >= 0) & (li < _NPC)
                        lidxv[q, pl.ds(g * 16, 16)] = jnp.where(ok, li, _DUMMY)
                sds = []
                for half in range(2):
                    for q in (2 * half, 2 * half + 1):
                        gds[q].wait()
                    for q in (2 * half, 2 * half + 1):
                        def _scale(g, carry2, _q=q, _b=b):
                            ew16 = ewb[_b, _q, pl.ds(g * 16, 16)]
                            for l in range(16):
                                k = g * 16 + l
                                w16 = lax.gather(
                                    ew16, jnp.full((16, 1), l, jnp.int32),
                                    lax.GatherDimensionNumbers(
                                        offset_dims=(),
                                        collapsed_slice_dims=(0,),
                                        start_index_map=(0,)),
                                    (1,),
                                    mode=lax.GatherScatterMode.PROMISE_IN_BOUNDS)
                                for f in range(3):
                                    sl = pl.ds(f * 16, 16)
                                    rows[_q, k, sl] = rows[_q, k, sl] * w16
                            return carry2
                        lax.fori_loop(0, 8, _scale, 0)
                    for q in (2 * half, 2 * half + 1):
                        sds.append(pltpu.async_copy(rows.at[q],
                                                    y_sh.at[lidxv.at[q]],
                                                    ssem, add=True))
                nb = tbase + (j + 2) * 4
                pltpu.async_copy(src_hbm.at[pl.ds(nb, 4)], sv.at[b], sem)
                pltpu.async_copy(dst_hbm.at[pl.ds(nb, 4)], dv.at[b], sem)
                pltpu.async_copy(ew_hbm.at[pl.ds(nb, 4)], ewb.at[b], sem)
                for d in sds:
                    d.wait()
            return carry
        lax.fori_loop(0, _TCH_C // 8, _iter, 0)

        for b in range(2):
            semb = msem0 if b == 0 else msem1
            pltpu.make_async_copy(src_hbm.at[pl.ds(0, 4)], sv.at[b], semb).wait()
            pltpu.make_async_copy(dst_hbm.at[pl.ds(0, 4)], dv.at[b], semb).wait()
            pltpu.make_async_copy(ew_hbm.at[pl.ds(0, 4)], ewb.at[b], semb).wait()

        plsc.subcore_barrier()
        for t in range(12):
            pltpu.sync_copy(y_sh.at[pl.ds(ybase + t * 128, 128)], rows.at[0])
            pltpu.sync_copy(rows.at[0],
                            out_hbm.at[c, p, pl.ds(ybase + t * 128, 128)])
        pltpu.sync_copy(y_sh.at[pl.ds(ybase + 1536, 64)],
                        rows.at[0, pl.ds(0, 64)])
        pltpu.sync_copy(rows.at[0, pl.ds(0, 64)],
                        out_hbm.at[c, p, pl.ds(ybase + 1536, 64)])
        plsc.subcore_barrier()


@functools.partial(
    pl.kernel,
    out_type=jax.ShapeDtypeStruct((2, 2, _NPC, _FH), jnp.float32),
    mesh=_mesh,
    scratch_types=[
        pltpu.VMEM_SHARED((_YROWS, _FH), jnp.float32),
        pltpu.VMEM((2, 4, 128), jnp.int32),
        pltpu.VMEM((2, 4, 128), jnp.int32),
        pltpu.VMEM((2, 4, 128), jnp.float32),
        pltpu.VMEM((4, 128), jnp.int32),
        pltpu.VMEM((4, 128, _FH), jnp.float32),
        pltpu.VMEM((32, _FH), jnp.float32),
        pltpu.SemaphoreType.DMA,
        pltpu.SemaphoreType.DMA,
        pltpu.SemaphoreType.DMA,
        pltpu.SemaphoreType.DMA,
        pltpu.SemaphoreType.DMA,
    ],
    compiler_params=pltpu.CompilerParams(use_tc_tiling_on_sc=False),
)
def _agg_kernel(src_hbm, dst_hbm, ew_hbm, xs0_hbm, xs1_hbm, out_hbm,
                y_sh, sv, dv, ewb, lidxv, rows, zrows, gsemA, gsemB, ssem,
                msem0, msem1):
    _agg_body(src_hbm, dst_hbm, ew_hbm, xs0_hbm, xs1_hbm, out_hbm,
              y_sh, sv, dv, ewb, lidxv, rows, zrows, gsemA, gsemB, ssem,
              msem0, msem1)


def _scale_body(degp_ref, xh0_ref, xh1_ref, xs0_ref, xs1_ref):
    d = degp_ref[0, :] + degp_ref[1, :] + 1.0
    d_safe = jnp.where(d > 0, d, 1.0)
    dis = jnp.where(d > 0, lax.rsqrt(d_safe), 0.0)[:, None]
    xs0_ref[...] = xh0_ref[...] * dis
    xs1_ref[...] = xh1_ref[...] * dis


def _prescale_stage(degp, xh0, xh1):
    return pl.pallas_call(
        _scale_body,
        grid=(_NPAD // _BLK,),
        in_specs=[
            pl.BlockSpec((2, _BLK), lambda i: (0, i)),
            pl.BlockSpec((_BLK, _FH), lambda i: (i, 0)),
            pl.BlockSpec((_BLK, _FH), lambda i: (i, 0)),
        ],
        out_specs=[
            pl.BlockSpec((_BLK, _FH), lambda i: (i, 0)),
            pl.BlockSpec((_BLK, _FH), lambda i: (i, 0)),
        ],
        out_shape=[
            jax.ShapeDtypeStruct((_NPAD, _FH), jnp.float32),
            jax.ShapeDtypeStruct((_NPAD, _FH), jnp.float32),
        ],
    )(degp, xh0, xh1)


def _dense_body(y_ref, xs0_ref, xs1_ref, degp_ref, mzb_ref, mhb_ref, czb_ref,
                chb_ref, probs_ref, clsw_ref, clsb_ref, out_ref):
    d = degp_ref[0, :] + degp_ref[1, :] + 1.0
    d_safe = jnp.where(d > 0, d, 1.0)
    dis = jnp.where(d > 0, lax.rsqrt(d_safe), 0.0)[:, None]
    xs = jnp.concatenate([xs0_ref[...], xs1_ref[...]], axis=1)
    y = jnp.concatenate([y_ref[0, 0], y_ref[0, 1]], axis=1)
    yt = (y + xs) * dis
    u_z = jnp.dot(yt, mzb_ref[...], preferred_element_type=jnp.float32) + czb_ref[...]
    u_h = jnp.dot(yt, mhb_ref[...], preferred_element_type=jnp.float32) + chb_ref[...]
    acc = jnp.zeros((_BLK, _H), dtype=jnp.float32)
    probs = probs_ref[...]
    for t in range(_P):
        z = jax.nn.sigmoid(u_z[:, t * _H:(t + 1) * _H])
        htil = jnp.tanh(u_h[:, t * _H:(t + 1) * _H])
        acc = acc + probs[0, t] * (1.0 - z) * htil
    h = jnp.maximum(acc, 0.0)
    out_ref[...] = jnp.dot(h, clsw_ref[...], preferred_element_type=jnp.float32) + clsb_ref[...]


def _dense_stage(y_raw, xs0, xs1, degp, mz_big, mh_big, cz_big, ch_big, probs,
                 cls_W, cls_b):
    return pl.pallas_call(
        _dense_body,
        grid=(_NPAD // _BLK,),
        in_specs=[
            pl.BlockSpec((1, 2, _BLK, _FH), lambda i: (i // 25, 0, i % 25, 0)),
            pl.BlockSpec((_BLK, _FH), lambda i: (i, 0)),
            pl.BlockSpec((_BLK, _FH), lambda i: (i, 0)),
            pl.BlockSpec((2, _BLK), lambda i: (0, i)),
            pl.BlockSpec((_FT, _P * _H), lambda i: (0, 0)),
            pl.BlockSpec((_FT, _P * _H), lambda i: (0, 0)),
            pl.BlockSpec((1, _P * _H), lambda i: (0, 0)),
            pl.BlockSpec((1, _P * _H), lambda i: (0, 0)),
            pl.BlockSpec((1, _P), lambda i: (0, 0)),
            pl.BlockSpec((_H, _P), lambda i: (0, 0)),
            pl.BlockSpec((1, _P), lambda i: (0, 0)),
        ],
        out_specs=pl.BlockSpec((_BLK, _P), lambda i: (i, 0)),
        out_shape=jax.ShapeDtypeStruct((_NPAD, _P), jnp.float32),
    )(y_raw, xs0, xs1, degp, mz_big, mh_big, cz_big, ch_big, probs, cls_W, cls_b)


@jax.jit
def _run(x, edge_index, edge_weight, attention, W_z, b_z, LW_z, Lb_z,
         W_h, b_h, LW_h, Lb_h, cls_W, cls_b):
    n = x.shape[0]

    probs = jax.nn.softmax(attention)
    A_z = LW_z[:_H]
    A_h = LW_h[:_H]
    M_z = W_z @ A_z
    M_h = W_h @ A_h
    c_z = b_z @ A_z + Lb_z
    c_h = b_h @ A_h + Lb_h
    eye = jnp.eye(_P, dtype=jnp.float32)
    mz_big = jnp.einsum('fj,tu->ftuj', M_z, eye).reshape(_FT, _P * _H)
    mh_big = jnp.einsum('fj,tu->ftuj', M_h, eye).reshape(_FT, _P * _H)
    cz_big = jnp.tile(c_z, _P)[None, :]
    ch_big = jnp.tile(c_h, _P)[None, :]

    x_flat = x.reshape(n, _FT)
    x_p = jnp.pad(x_flat, ((0, _NPAD - n), (0, 0)))
    xh0 = x_p[:, :_FH]
    xh1 = x_p[:, _FH:]
    epad = _EROWS * 128 - _E
    src2d = jnp.pad(edge_index[0], (0, epad)).reshape(_EROWS, 128)
    dst2d = jnp.pad(edge_index[1], (0, epad)).reshape(_EROWS, 128)
    ew2d = jnp.pad(edge_weight, (0, epad)).reshape(_EROWS, 128)

    degp = _deg_kernel(dst2d, ew2d).reshape(2, _NPAD)

    xs0, xs1 = _prescale_stage(degp, xh0, xh1)

    yout = _agg_kernel(src2d, dst2d, ew2d, xs0, xs1)

    out = _dense_stage(yout, xs0, xs1, degp, mz_big, mh_big, cz_big, ch_big,
                       probs[None, :], cls_W, cls_b[None, :])
    return out[:n]


def kernel(x, edge_index, edge_weight, attention, W_z, b_z, LW_z, Lb_z,
           W_r, b_r, LW_r, Lb_r, W_h, b_h, LW_h, Lb_h, cls_W, cls_b):
    return _run(x, edge_index, edge_weight, attention, W_z, b_z, LW_z, Lb_z,
                W_h, b_h, LW_h, Lb_h, cls_W, cls_b)
